# baseline (speedup 1.0000x reference)
import functools

import jax
import jax.numpy as jnp
from jax import lax
from jax.experimental import pallas as pl
from jax.experimental.pallas import tpu as pltpu
from jax.experimental.pallas import tpu_sc as plsc

_NUM = 8
_NA = 5000
_NG = 16
_CHUNK = 1250
_VPT = 79
_ARR = _VPT * 16
_AWIN = 2512
_IWIN = 1264
_LN2 = 0.6931471805599453


def _vlog(x):
    bits = lax.bitcast_convert_type(x, jnp.int32)
    e = lax.shift_right_logical(bits, 23) - 127
    m = lax.bitcast_convert_type(
        jnp.bitwise_or(jnp.bitwise_and(bits, 0x007FFFFF), 0x3F800000),
        jnp.float32)
    big = m > 1.4142135623730951
    m = jnp.where(big, m * 0.5, m)
    e = jnp.where(big, e + 1, e)
    t = (m - 1.0) / (m + 1.0)
    t2 = t * t
    p = 1.0 + t2 * (1.0 / 3.0 + t2 * (1.0 / 5.0 + t2 * (1.0 / 7.0 + t2 * (1.0 / 9.0))))
    return e.astype(jnp.float32) * _LN2 + 2.0 * t * p


def _sc_body(anch_h, loc_h, ign_h, tg_h, out_h,
             av_v, lv_v, ig_v, t_v,
             ac_v, aw_v, l0_v, l1_v, ign_v,
             as_v, ae_v, alen_v, bov_v, bidx_v, bp_v,
             stage_v, tmp_v, mc_v, mw_v, res_v, sh):
    c = lax.axis_index("c")
    s = lax.axis_index("s")
    batch = c * 4 + s // 4
    chunk = s % 4
    anchor0 = chunk * _CHUNK

    a_off = batch * (2 * _NA) + chunk * (2 * _CHUNK)
    a_start = jnp.minimum((a_off // 8) * 8, _NUM * 2 * _NA - _AWIN)
    a_shift = a_off - a_start
    i_off = batch * _NA + chunk * _CHUNK
    i_start = jnp.minimum((i_off // 8) * 8, _NUM * _NA - _IWIN)
    i_shift = i_off - i_start
    pltpu.sync_copy(anch_h.at[pl.ds(a_start, _AWIN)], av_v)
    pltpu.sync_copy(loc_h.at[pl.ds(a_start, _AWIN)], lv_v)
    pltpu.sync_copy(ign_h.at[pl.ds(i_start, _IWIN)], ig_v)
    pltpu.sync_copy(tg_h.at[pl.ds(batch * (3 * _NG), 3 * _NG)], t_v)

    lane = lax.iota(jnp.int32, 16)
    neg1 = jnp.full((16,), -1.0, jnp.float32)
    zeroi = jnp.zeros((16,), jnp.int32)

    def setup(i, _):
        sl = pl.ds(i * 16, 16)
        k = i * 16 + lane
        valid = k < _CHUNK
        idx2 = jnp.minimum(a_shift + 2 * k, _AWIN - 2)
        ac = plsc.load_gather(av_v, [idx2])
        aw = plsc.load_gather(av_v, [idx2 + 1])
        l0 = plsc.load_gather(lv_v, [idx2])
        l1 = plsc.load_gather(lv_v, [idx2 + 1])
        ign = ig_v[pl.ds(i_shift + i * 16, 16)]
        ac = jnp.where(valid, ac, -10.0)
        aw = jnp.where(valid, aw, 1.0)
        ign = jnp.where(valid, ign, 1)
        ac_v[sl] = ac
        aw_v[sl] = aw
        l0_v[sl] = l0
        l1_v[sl] = l1
        ign_v[sl] = ign
        a_s = ac - aw / 2.0
        a_e = ac + aw / 2.0
        as_v[sl] = a_s
        ae_v[sl] = a_e
        alen_v[sl] = a_e - a_s
        bov_v[sl] = neg1
        bidx_v[sl] = zeroi
        return 0

    lax.fori_loop(0, _VPT, setup, 0)

    def per_g(g, carry):
        pk_i, pk_x = carry
        gidx = jnp.broadcast_to(g, (16,))
        gsb = plsc.load_gather(t_v, [3 * gidx])
        geb = plsc.load_gather(t_v, [3 * gidx + 1])
        glenb = geb - gsb

        def inner(i, cr):
            gm, gi = cr
            sl = pl.ds(i * 16, 16)
            a_s = as_v[sl]
            a_e = ae_v[sl]
            alen = alen_v[sl]
            inter = jnp.maximum(jnp.minimum(geb, a_e) - jnp.maximum(gsb, a_s), 0.0)
            union = jnp.maximum(glenb + alen - inter, 1e-10)
            iou = inter / union
            bov = bov_v[sl]
            upd = iou > bov
            bov_v[sl] = jnp.where(upd, iou, bov)
            bidx_v[sl] = jnp.where(upd, gidx, bidx_v[sl])
            aidx = anchor0 + i * 16 + lane
            gu = iou > gm
            gm = jnp.where(gu, iou, gm)
            gi = jnp.where(gu, aidx, gi)
            return gm, gi

        gm, gi = lax.fori_loop(0, _VPT, inner, (neg1, zeroi))
        m = jnp.max(gm)
        cand = jnp.where(gm == m, gi, jnp.int32(2 ** 30))
        mi = jnp.min(cand)
        lm = lane == g
        pk_i = jnp.where(lm, m, pk_i)
        pk_x = jnp.where(lm, mi, pk_x)
        return pk_i, pk_x

    pk_i, pk_x = lax.fori_loop(0, _NG, per_g,
                               (jnp.zeros((16,), jnp.float32), zeroi))

    stage_v[pl.ds(0, 16)] = pk_i
    stage_v[pl.ds(16, 16)] = lax.bitcast_convert_type(pk_x, jnp.float32)
    pltpu.sync_copy(stage_v, sh.at[pl.ds(s * 32, 32)])
    plsc.subcore_barrier()
    s0 = (s // 4) * 4
    pltpu.sync_copy(sh.at[pl.ds(s0 * 32, 32)], tmp_v)
    cur_i = tmp_v[pl.ds(0, 16)]
    cur_x = lax.bitcast_convert_type(tmp_v[pl.ds(16, 16)], jnp.int32)
    for cc in range(1, 4):
        pltpu.sync_copy(sh.at[pl.ds((s0 + cc) * 32, 32)], tmp_v)
        vi = tmp_v[pl.ds(0, 16)]
        vx = lax.bitcast_convert_type(tmp_v[pl.ds(16, 16)], jnp.int32)
        u = vi > cur_i
        cur_i = jnp.where(u, vi, cur_i)
        cur_x = jnp.where(u, vx, cur_x)
    plsc.subcore_barrier()

    bp_v[...] = cur_x
    two_f = jnp.full((16,), 2.0, jnp.float32)

    def override(g, _):
        gidx = jnp.broadcast_to(g, (16,))
        bpg = plsc.load_gather(bp_v, [gidx])
        loc = bpg - anchor0
        mask = jnp.logical_and(lane == g,
                               jnp.logical_and(loc >= 0, loc < _CHUNK))
        li = jnp.clip(loc, 0, _CHUNK - 1)
        plsc.store_scatter(bov_v, [li], two_f, mask=mask)
        plsc.store_scatter(bidx_v, [li], gidx, mask=mask)
        return 0

    lax.fori_loop(0, _NG, override, 0)

    gs_vec = plsc.load_gather(t_v, [3 * lane])
    ge_vec = plsc.load_gather(t_v, [3 * lane + 1])
    mc_v[...] = (gs_vec + ge_vec) / 2.0
    mw_v[...] = ge_vec - gs_vec

    def p2(i, carry):
        ls, cs = carry
        sl = pl.ds(i * 16, 16)
        bov = bov_v[sl]
        bidx = bidx_v[sl]
        ign = ign_v[sl]
        ac = ac_v[sl]
        aw = aw_v[sl]
        l0 = l0_v[sl]
        l1 = l1_v[sl]
        p = jnp.logical_and(bov >= 0.5, ign == 0)
        mc = plsc.load_gather(mc_v, [bidx])
        mw = plsc.load_gather(mw_v, [bidx])
        lc = (mc - ac) / (0.1 * aw)
        r = jnp.maximum(mw / aw, 1e-10)
        lw = _vlog(r) / 0.2
        d0 = l0 - lc
        d1 = l1 - lw
        a0 = jnp.abs(d0)
        a1 = jnp.abs(d1)
        s0_ = jnp.where(a0 < 1.0, 0.5 * a0 * a0, a0 - 0.5)
        s1_ = jnp.where(a1 < 1.0, 0.5 * a1 * a1, a1 - 0.5)
        ls = ls + jnp.where(p, s0_ + s1_, 0.0)
        cs = cs + jnp.where(p, 1.0, 0.0)
        return ls, cs

    zero16 = jnp.zeros((16,), jnp.float32)
    ls, cs = lax.fori_loop(0, _VPT, p2, (zero16, zero16))
    lsum = jnp.sum(ls)
    csum = jnp.sum(cs)
    lane_f = lane
    stage_v[pl.ds(0, 16)] = jnp.where(lane_f == 0, lsum,
                                      jnp.where(lane_f == 1, csum, 0.0))
    pltpu.sync_copy(stage_v, sh.at[pl.ds(s * 32, 32)])
    plsc.subcore_barrier()

    @pl.when(s == 0)
    def _():
        acc = jnp.zeros((16,), jnp.float32)
        for k in range(16):
            pltpu.sync_copy(sh.at[pl.ds(k * 32, 32)], tmp_v)
            acc = acc + tmp_v[pl.ds(0, 16)]
        res_v[...] = acc
        pltpu.sync_copy(res_v, out_h.at[pl.ds(c * 16, 16)])


@functools.partial(
    pl.kernel,
    mesh=plsc.VectorSubcoreMesh(core_axis_name="c", subcore_axis_name="s"),
    out_type=jax.ShapeDtypeStruct((32,), jnp.float32),
    compiler_params=pltpu.CompilerParams(needs_layout_passes=False),
    scratch_types=[
        pltpu.VMEM((_AWIN,), jnp.float32),
        pltpu.VMEM((_AWIN,), jnp.float32),
        pltpu.VMEM((_IWIN,), jnp.int32),
        pltpu.VMEM((3 * _NG,), jnp.float32),
        pltpu.VMEM((_ARR,), jnp.float32),
        pltpu.VMEM((_ARR,), jnp.float32),
        pltpu.VMEM((_ARR,), jnp.float32),
        pltpu.VMEM((_ARR,), jnp.float32),
        pltpu.VMEM((_ARR,), jnp.int32),
        pltpu.VMEM((_ARR,), jnp.float32),
        pltpu.VMEM((_ARR,), jnp.float32),
        pltpu.VMEM((_ARR,), jnp.float32),
        pltpu.VMEM((_ARR,), jnp.float32),
        pltpu.VMEM((_ARR,), jnp.int32),
        pltpu.VMEM((16,), jnp.int32),
        pltpu.VMEM((32,), jnp.float32),
        pltpu.VMEM((32,), jnp.float32),
        pltpu.VMEM((_NG,), jnp.float32),
        pltpu.VMEM((_NG,), jnp.float32),
        pltpu.VMEM((16,), jnp.float32),
        pltpu.VMEM_SHARED((512,), jnp.float32),
    ],
)
def _fgd_sc(*refs):
    _sc_body(*refs)


def kernel(loc_pred, conf_pred, refined_anchors, ignore_flags_refined_anchor, targets):
    del conf_pred
    out = _fgd_sc(refined_anchors.reshape(-1), loc_pred.reshape(-1),
                  ignore_flags_refined_anchor.reshape(-1), targets.reshape(-1))
    return (out[0] + out[16]) / (out[1] + out[17])


# speedup vs baseline: 2.1600x; 2.1600x over previous
import functools

import jax
import jax.numpy as jnp
from jax import lax
from jax.experimental import pallas as pl
from jax.experimental.pallas import tpu as pltpu
from jax.experimental.pallas import tpu_sc as plsc

_NUM = 8
_NA = 5000
_NG = 16
_PAD_NA = 5120
_CHUNK = 1280
_VPT = _CHUNK // 16
_LN2 = 0.6931471805599453


def _vlog(x):
    bits = lax.bitcast_convert_type(x, jnp.int32)
    e = lax.shift_right_logical(bits, 23) - 127
    m = lax.bitcast_convert_type(
        jnp.bitwise_or(jnp.bitwise_and(bits, 0x007FFFFF), 0x3F800000),
        jnp.float32)
    big = m > 1.4142135623730951
    m = jnp.where(big, m * 0.5, m)
    e = jnp.where(big, e + 1, e)
    t = (m - 1.0) / (m + 1.0)
    t2 = t * t
    p = 1.0 + t2 * (1.0 / 3.0 + t2 * (1.0 / 5.0 + t2 * (1.0 / 7.0 + t2 * (1.0 / 9.0))))
    return e.astype(jnp.float32) * _LN2 + 2.0 * t * p


_SEG = _NUM * _PAD_NA


def _sc_body(big_h, out_h,
             ac_v, aw_v, l0_v, l1_v, igf_v, t_v,
             as_v, ae_v, alen_v, bov_v, bidx_v, bp_v,
             stage_v, tmp_v, mc_v, mw_v, res_v, sh):
    c = lax.axis_index("c")
    s = lax.axis_index("s")
    batch = c * 4 + s // 4
    chunk = s % 4
    base = batch * _PAD_NA + chunk * _CHUNK
    anchor0 = chunk * _CHUNK

    pltpu.sync_copy(big_h.at[pl.ds(base, _CHUNK)], ac_v)
    pltpu.sync_copy(big_h.at[pl.ds(_SEG + base, _CHUNK)], aw_v)
    pltpu.sync_copy(big_h.at[pl.ds(2 * _SEG + base, _CHUNK)], l0_v)
    pltpu.sync_copy(big_h.at[pl.ds(3 * _SEG + base, _CHUNK)], l1_v)
    pltpu.sync_copy(big_h.at[pl.ds(4 * _SEG + base, _CHUNK)], igf_v)
    pltpu.sync_copy(big_h.at[pl.ds(5 * _SEG + batch * 48, 48)], t_v)

    lane = lax.iota(jnp.int32, 16)
    neg1 = jnp.full((16,), -1.0, jnp.float32)
    zeroi = jnp.zeros((16,), jnp.int32)

    def setup(i, _):
        sl = pl.ds(i * 16, 16)
        ac = ac_v[sl]
        aw = aw_v[sl]
        a_s = ac - aw / 2.0
        a_e = ac + aw / 2.0
        as_v[sl] = a_s
        ae_v[sl] = a_e
        alen_v[sl] = a_e - a_s
        bov_v[sl] = neg1
        bidx_v[sl] = zeroi
        return 0

    lax.fori_loop(0, _VPT, setup, 0)

    def per_g(g, carry):
        pk_i, pk_x = carry
        gidx = jnp.broadcast_to(g, (16,))
        gsb = plsc.load_gather(t_v, [3 * gidx])
        geb = plsc.load_gather(t_v, [3 * gidx + 1])
        glenb = geb - gsb

        def inner(i, cr):
            gm, gi = cr
            sl = pl.ds(i * 16, 16)
            a_s = as_v[sl]
            a_e = ae_v[sl]
            alen = alen_v[sl]
            inter = jnp.maximum(jnp.minimum(geb, a_e) - jnp.maximum(gsb, a_s), 0.0)
            union = jnp.maximum(glenb + alen - inter, 1e-10)
            iou = inter / union
            bov = bov_v[sl]
            upd = iou > bov
            bov_v[sl] = jnp.where(upd, iou, bov)
            bidx_v[sl] = jnp.where(upd, gidx, bidx_v[sl])
            aidx = anchor0 + i * 16 + lane
            gu = iou > gm
            gm = jnp.where(gu, iou, gm)
            gi = jnp.where(gu, aidx, gi)
            return gm, gi

        gm, gi = lax.fori_loop(0, _VPT, inner, (neg1, zeroi))
        m = jnp.max(gm)
        cand = jnp.where(gm == m, gi, jnp.int32(2 ** 30))
        mi = jnp.min(cand)
        lm = lane == g
        pk_i = jnp.where(lm, m, pk_i)
        pk_x = jnp.where(lm, mi, pk_x)
        return pk_i, pk_x

    pk_i, pk_x = lax.fori_loop(0, _NG, per_g,
                               (jnp.zeros((16,), jnp.float32), zeroi))

    stage_v[pl.ds(0, 16)] = pk_i
    stage_v[pl.ds(16, 16)] = lax.bitcast_convert_type(pk_x, jnp.float32)
    pltpu.sync_copy(stage_v, sh.at[pl.ds(s * 32, 32)])
    plsc.subcore_barrier()
    s0 = (s // 4) * 4
    pltpu.sync_copy(sh.at[pl.ds(s0 * 32, 32)], tmp_v)
    cur_i = tmp_v[pl.ds(0, 16)]
    cur_x = lax.bitcast_convert_type(tmp_v[pl.ds(16, 16)], jnp.int32)
    for cc in range(1, 4):
        pltpu.sync_copy(sh.at[pl.ds((s0 + cc) * 32, 32)], tmp_v)
        vi = tmp_v[pl.ds(0, 16)]
        vx = lax.bitcast_convert_type(tmp_v[pl.ds(16, 16)], jnp.int32)
        u = vi > cur_i
        cur_i = jnp.where(u, vi, cur_i)
        cur_x = jnp.where(u, vx, cur_x)
    plsc.subcore_barrier()

    bp_v[...] = cur_x
    two_f = jnp.full((16,), 2.0, jnp.float32)

    def override(g, _):
        gidx = jnp.broadcast_to(g, (16,))
        bpg = plsc.load_gather(bp_v, [gidx])
        loc = bpg - anchor0
        mask = jnp.logical_and(lane == g,
                               jnp.logical_and(loc >= 0, loc < _CHUNK))
        li = jnp.clip(loc, 0, _CHUNK - 1)
        plsc.store_scatter(bov_v, [li], two_f, mask=mask)
        plsc.store_scatter(bidx_v, [li], gidx, mask=mask)
        return 0

    lax.fori_loop(0, _NG, override, 0)

    gs_vec = plsc.load_gather(t_v, [3 * lane])
    ge_vec = plsc.load_gather(t_v, [3 * lane + 1])
    mc_v[...] = (gs_vec + ge_vec) / 2.0
    mw_v[...] = ge_vec - gs_vec

    def p2(i, carry):
        ls, cs = carry
        sl = pl.ds(i * 16, 16)
        bov = bov_v[sl]
        bidx = bidx_v[sl]
        igf = igf_v[sl]
        ac = ac_v[sl]
        aw = aw_v[sl]
        l0 = l0_v[sl]
        l1 = l1_v[sl]
        p = jnp.logical_and(bov >= 0.5, igf == 0.0)
        mc = plsc.load_gather(mc_v, [bidx])
        mw = plsc.load_gather(mw_v, [bidx])
        lc = (mc - ac) / (0.1 * aw)
        r = jnp.maximum(mw / aw, 1e-10)
        lw = _vlog(r) / 0.2
        d0 = l0 - lc
        d1 = l1 - lw
        a0 = jnp.abs(d0)
        a1 = jnp.abs(d1)
        s0_ = jnp.where(a0 < 1.0, 0.5 * a0 * a0, a0 - 0.5)
        s1_ = jnp.where(a1 < 1.0, 0.5 * a1 * a1, a1 - 0.5)
        ls = ls + jnp.where(p, s0_ + s1_, 0.0)
        cs = cs + jnp.where(p, 1.0, 0.0)
        return ls, cs

    zero16 = jnp.zeros((16,), jnp.float32)
    ls, cs = lax.fori_loop(0, _VPT, p2, (zero16, zero16))
    lsum = jnp.sum(ls)
    csum = jnp.sum(cs)
    stage_v[pl.ds(0, 16)] = jnp.where(lane == 0, lsum, jnp.where(lane == 1, csum, 0.0))
    pltpu.sync_copy(stage_v, sh.at[pl.ds(s * 32, 32)])
    plsc.subcore_barrier()

    @pl.when(s == 0)
    def _():
        acc = jnp.zeros((16,), jnp.float32)
        for k in range(16):
            pltpu.sync_copy(sh.at[pl.ds(k * 32, 32)], tmp_v)
            acc = acc + tmp_v[pl.ds(0, 16)]
        res_v[...] = acc
        pltpu.sync_copy(res_v, out_h.at[pl.ds(c * 16, 16)])


@functools.partial(
    pl.kernel,
    mesh=plsc.VectorSubcoreMesh(core_axis_name="c", subcore_axis_name="s"),
    out_type=jax.ShapeDtypeStruct((32,), jnp.float32),
    compiler_params=pltpu.CompilerParams(needs_layout_passes=False),
    scratch_types=[
        pltpu.VMEM((_CHUNK,), jnp.float32),
        pltpu.VMEM((_CHUNK,), jnp.float32),
        pltpu.VMEM((_CHUNK,), jnp.float32),
        pltpu.VMEM((_CHUNK,), jnp.float32),
        pltpu.VMEM((_CHUNK,), jnp.float32),
        pltpu.VMEM((3 * _NG,), jnp.float32),
        pltpu.VMEM((_CHUNK,), jnp.float32),
        pltpu.VMEM((_CHUNK,), jnp.float32),
        pltpu.VMEM((_CHUNK,), jnp.float32),
        pltpu.VMEM((_CHUNK,), jnp.float32),
        pltpu.VMEM((_CHUNK,), jnp.int32),
        pltpu.VMEM((16,), jnp.int32),
        pltpu.VMEM((32,), jnp.float32),
        pltpu.VMEM((32,), jnp.float32),
        pltpu.VMEM((_NG,), jnp.float32),
        pltpu.VMEM((_NG,), jnp.float32),
        pltpu.VMEM((16,), jnp.float32),
        pltpu.VMEM_SHARED((512,), jnp.float32),
    ],
)
def _fgd_sc(*refs):
    _sc_body(*refs)


def kernel(loc_pred, conf_pred, refined_anchors, ignore_flags_refined_anchor, targets):
    del conf_pred
    pad = _PAD_NA - _NA
    ac = jnp.pad(refined_anchors[..., 0], ((0, 0), (0, pad)), constant_values=-10.0)
    aw = jnp.pad(refined_anchors[..., 1], ((0, 0), (0, pad)), constant_values=1.0)
    l0 = jnp.pad(loc_pred[..., 0], ((0, 0), (0, pad)))
    l1 = jnp.pad(loc_pred[..., 1], ((0, 0), (0, pad)))
    igf = jnp.pad(ignore_flags_refined_anchor.astype(jnp.float32),
                  ((0, 0), (0, pad)), constant_values=1.0)
    big = jnp.concatenate([ac.reshape(-1), aw.reshape(-1), l0.reshape(-1),
                           l1.reshape(-1), igf.reshape(-1), targets.reshape(-1)])
    out = _fgd_sc(big)
    return (out[0] + out[16]) / (out[1] + out[17])


# speedup vs baseline: 2.2547x; 1.0438x over previous
import functools

import jax
import jax.numpy as jnp
from jax import lax
from jax.experimental import pallas as pl
from jax.experimental.pallas import tpu as pltpu
from jax.experimental.pallas import tpu_sc as plsc

_NUM = 8
_NA = 5000
_NG = 16
_PAD_NA = 5120
_CHUNK = 1280
_VPT = _CHUNK // 16
_LN2 = 0.6931471805599453


def _vlog(x):
    bits = lax.bitcast_convert_type(x, jnp.int32)
    e = lax.shift_right_logical(bits, 23) - 127
    m = lax.bitcast_convert_type(
        jnp.bitwise_or(jnp.bitwise_and(bits, 0x007FFFFF), 0x3F800000),
        jnp.float32)
    big = m > 1.4142135623730951
    m = jnp.where(big, m * 0.5, m)
    e = jnp.where(big, e + 1, e)
    t = (m - 1.0) / (m + 1.0)
    t2 = t * t
    p = 1.0 + t2 * (1.0 / 3.0 + t2 * (1.0 / 5.0 + t2 * (1.0 / 7.0 + t2 * (1.0 / 9.0))))
    return e.astype(jnp.float32) * _LN2 + 2.0 * t * p


_SEG = _NUM * _PAD_NA


def _sc_body(big_h, out_h,
             ac_v, aw_v, l0_v, l1_v, igf_v, t_v,
             as_v, ae_v, alen_v, bov_v, bidx_v, bp_v,
             stage_v, tmp_v, mc_v, mw_v, res_v, sh):
    c = lax.axis_index("c")
    s = lax.axis_index("s")
    batch = c * 4 + s // 4
    chunk = s % 4
    base = batch * _PAD_NA + chunk * _CHUNK
    anchor0 = chunk * _CHUNK

    pltpu.sync_copy(big_h.at[pl.ds(base, _CHUNK)], ac_v)
    pltpu.sync_copy(big_h.at[pl.ds(_SEG + base, _CHUNK)], aw_v)
    pltpu.sync_copy(big_h.at[pl.ds(2 * _SEG + base, _CHUNK)], l0_v)
    pltpu.sync_copy(big_h.at[pl.ds(3 * _SEG + base, _CHUNK)], l1_v)
    pltpu.sync_copy(big_h.at[pl.ds(4 * _SEG + base, _CHUNK)], igf_v)
    pltpu.sync_copy(big_h.at[pl.ds(5 * _SEG + batch * 48, 48)], t_v)

    lane = lax.iota(jnp.int32, 16)
    neg1 = jnp.full((16,), -1.0, jnp.float32)
    zeroi = jnp.zeros((16,), jnp.int32)

    @plsc.parallel_loop(0, _VPT, unroll=2)
    def _setup(i):
        sl = pl.ds(i * 16, 16)
        ac = ac_v[sl]
        aw = aw_v[sl]
        a_s = ac - aw / 2.0
        a_e = ac + aw / 2.0
        as_v[sl] = a_s
        ae_v[sl] = a_e
        alen_v[sl] = a_e - a_s
        bov_v[sl] = neg1
        bidx_v[sl] = zeroi

    def per_g(g, carry):
        pk_i, pk_x = carry
        gidx = jnp.broadcast_to(g, (16,))
        gsb = plsc.load_gather(t_v, [3 * gidx])
        geb = plsc.load_gather(t_v, [3 * gidx + 1])
        glenb = geb - gsb

        def inner(i, cr):
            gm, gi = cr
            sl = pl.ds(i * 16, 16)
            a_s = as_v[sl]
            a_e = ae_v[sl]
            alen = alen_v[sl]
            inter = jnp.maximum(jnp.minimum(geb, a_e) - jnp.maximum(gsb, a_s), 0.0)
            union = jnp.maximum(glenb + alen - inter, 1e-10)
            iou = inter / union
            bov = bov_v[sl]
            upd = iou > bov
            bov_v[sl] = jnp.where(upd, iou, bov)
            bidx_v[sl] = jnp.where(upd, gidx, bidx_v[sl])
            aidx = anchor0 + i * 16 + lane
            gu = iou > gm
            gm = jnp.where(gu, iou, gm)
            gi = jnp.where(gu, aidx, gi)
            return gm, gi

        gm, gi = plsc.parallel_loop(0, _VPT, 1, unroll=4,
                                    carry=(neg1, zeroi))(inner)
        m = jnp.max(gm)
        cand = jnp.where(gm == m, gi, jnp.int32(2 ** 30))
        mi = jnp.min(cand)
        lm = lane == g
        pk_i = jnp.where(lm, m, pk_i)
        pk_x = jnp.where(lm, mi, pk_x)
        return pk_i, pk_x

    pk_i, pk_x = lax.fori_loop(0, _NG, per_g,
                               (jnp.zeros((16,), jnp.float32), zeroi))

    stage_v[pl.ds(0, 16)] = pk_i
    stage_v[pl.ds(16, 16)] = lax.bitcast_convert_type(pk_x, jnp.float32)
    pltpu.sync_copy(stage_v, sh.at[pl.ds(s * 32, 32)])
    plsc.subcore_barrier()
    s0 = (s // 4) * 4
    pltpu.sync_copy(sh.at[pl.ds(s0 * 32, 32)], tmp_v)
    cur_i = tmp_v[pl.ds(0, 16)]
    cur_x = lax.bitcast_convert_type(tmp_v[pl.ds(16, 16)], jnp.int32)
    for cc in range(1, 4):
        pltpu.sync_copy(sh.at[pl.ds((s0 + cc) * 32, 32)], tmp_v)
        vi = tmp_v[pl.ds(0, 16)]
        vx = lax.bitcast_convert_type(tmp_v[pl.ds(16, 16)], jnp.int32)
        u = vi > cur_i
        cur_i = jnp.where(u, vi, cur_i)
        cur_x = jnp.where(u, vx, cur_x)
    plsc.subcore_barrier()

    bp_v[...] = cur_x
    two_f = jnp.full((16,), 2.0, jnp.float32)

    def override(g, _):
        gidx = jnp.broadcast_to(g, (16,))
        bpg = plsc.load_gather(bp_v, [gidx])
        loc = bpg - anchor0
        mask = jnp.logical_and(lane == g,
                               jnp.logical_and(loc >= 0, loc < _CHUNK))
        li = jnp.clip(loc, 0, _CHUNK - 1)
        plsc.store_scatter(bov_v, [li], two_f, mask=mask)
        plsc.store_scatter(bidx_v, [li], gidx, mask=mask)
        return 0

    lax.fori_loop(0, _NG, override, 0)

    gs_vec = plsc.load_gather(t_v, [3 * lane])
    ge_vec = plsc.load_gather(t_v, [3 * lane + 1])
    mc_v[...] = (gs_vec + ge_vec) / 2.0
    mw_v[...] = ge_vec - gs_vec

    def p2(i, carry):
        ls, cs = carry
        sl = pl.ds(i * 16, 16)
        bov = bov_v[sl]
        bidx = bidx_v[sl]
        igf = igf_v[sl]
        ac = ac_v[sl]
        aw = aw_v[sl]
        l0 = l0_v[sl]
        l1 = l1_v[sl]
        p = jnp.logical_and(bov >= 0.5, igf == 0.0)
        mc = plsc.load_gather(mc_v, [bidx])
        mw = plsc.load_gather(mw_v, [bidx])
        lc = (mc - ac) / (0.1 * aw)
        r = jnp.maximum(mw / aw, 1e-10)
        lw = _vlog(r) / 0.2
        d0 = l0 - lc
        d1 = l1 - lw
        a0 = jnp.abs(d0)
        a1 = jnp.abs(d1)
        s0_ = jnp.where(a0 < 1.0, 0.5 * a0 * a0, a0 - 0.5)
        s1_ = jnp.where(a1 < 1.0, 0.5 * a1 * a1, a1 - 0.5)
        ls = ls + jnp.where(p, s0_ + s1_, 0.0)
        cs = cs + jnp.where(p, 1.0, 0.0)
        return ls, cs

    zero16 = jnp.zeros((16,), jnp.float32)
    ls, cs = plsc.parallel_loop(0, _VPT, 1, unroll=2,
                                carry=(zero16, zero16))(p2)
    lsum = jnp.sum(ls)
    csum = jnp.sum(cs)
    stage_v[pl.ds(0, 16)] = jnp.where(lane == 0, lsum, jnp.where(lane == 1, csum, 0.0))
    pltpu.sync_copy(stage_v, sh.at[pl.ds(s * 32, 32)])
    plsc.subcore_barrier()

    @pl.when(s == 0)
    def _():
        acc = jnp.zeros((16,), jnp.float32)
        for k in range(16):
            pltpu.sync_copy(sh.at[pl.ds(k * 32, 32)], tmp_v)
            acc = acc + tmp_v[pl.ds(0, 16)]
        res_v[...] = acc
        pltpu.sync_copy(res_v, out_h.at[pl.ds(c * 16, 16)])


@functools.partial(
    pl.kernel,
    mesh=plsc.VectorSubcoreMesh(core_axis_name="c", subcore_axis_name="s"),
    out_type=jax.ShapeDtypeStruct((32,), jnp.float32),
    compiler_params=pltpu.CompilerParams(needs_layout_passes=False),
    scratch_types=[
        pltpu.VMEM((_CHUNK,), jnp.float32),
        pltpu.VMEM((_CHUNK,), jnp.float32),
        pltpu.VMEM((_CHUNK,), jnp.float32),
        pltpu.VMEM((_CHUNK,), jnp.float32),
        pltpu.VMEM((_CHUNK,), jnp.float32),
        pltpu.VMEM((3 * _NG,), jnp.float32),
        pltpu.VMEM((_CHUNK,), jnp.float32),
        pltpu.VMEM((_CHUNK,), jnp.float32),
        pltpu.VMEM((_CHUNK,), jnp.float32),
        pltpu.VMEM((_CHUNK,), jnp.float32),
        pltpu.VMEM((_CHUNK,), jnp.int32),
        pltpu.VMEM((16,), jnp.int32),
        pltpu.VMEM((32,), jnp.float32),
        pltpu.VMEM((32,), jnp.float32),
        pltpu.VMEM((_NG,), jnp.float32),
        pltpu.VMEM((_NG,), jnp.float32),
        pltpu.VMEM((16,), jnp.float32),
        pltpu.VMEM_SHARED((512,), jnp.float32),
    ],
)
def _fgd_sc(*refs):
    _sc_body(*refs)


def kernel(loc_pred, conf_pred, refined_anchors, ignore_flags_refined_anchor, targets):
    del conf_pred
    pad = _PAD_NA - _NA
    ac = jnp.pad(refined_anchors[..., 0], ((0, 0), (0, pad)), constant_values=-10.0)
    aw = jnp.pad(refined_anchors[..., 1], ((0, 0), (0, pad)), constant_values=1.0)
    l0 = jnp.pad(loc_pred[..., 0], ((0, 0), (0, pad)))
    l1 = jnp.pad(loc_pred[..., 1], ((0, 0), (0, pad)))
    igf = jnp.pad(ignore_flags_refined_anchor.astype(jnp.float32),
                  ((0, 0), (0, pad)), constant_values=1.0)
    big = jnp.concatenate([ac.reshape(-1), aw.reshape(-1), l0.reshape(-1),
                           l1.reshape(-1), igf.reshape(-1), targets.reshape(-1)])
    out = _fgd_sc(big)
    return (out[0] + out[16]) / (out[1] + out[17])


# speedup vs baseline: 2.2588x; 1.0018x over previous
import functools

import jax
import jax.numpy as jnp
from jax import lax
from jax.experimental import pallas as pl
from jax.experimental.pallas import tpu as pltpu
from jax.experimental.pallas import tpu_sc as plsc

_NUM = 8
_NA = 5000
_NG = 16
_PAD_NA = 5120
_CHUNK = 1280
_VPT = _CHUNK // 16
_LN2 = 0.6931471805599453


def _vlog(x):
    bits = lax.bitcast_convert_type(x, jnp.int32)
    e = lax.shift_right_logical(bits, 23) - 127
    m = lax.bitcast_convert_type(
        jnp.bitwise_or(jnp.bitwise_and(bits, 0x007FFFFF), 0x3F800000),
        jnp.float32)
    big = m > 1.4142135623730951
    m = jnp.where(big, m * 0.5, m)
    e = jnp.where(big, e + 1, e)
    t = (m - 1.0) / (m + 1.0)
    t2 = t * t
    p = 1.0 + t2 * (1.0 / 3.0 + t2 * (1.0 / 5.0 + t2 * (1.0 / 7.0 + t2 * (1.0 / 9.0))))
    return e.astype(jnp.float32) * _LN2 + 2.0 * t * p


_SEG = _NUM * _PAD_NA


def _sc_body(big_h, out_h,
             ac_v, aw_v, l0_v, l1_v, igf_v, t_v,
             as_v, ae_v, alen_v, bov_v, bidx_v, bp_v,
             stage_v, tmp_v, mc_v, mw_v, res_v, sh):
    c = lax.axis_index("c")
    s = lax.axis_index("s")
    batch = c * 4 + s // 4
    chunk = s % 4
    base = batch * _PAD_NA + chunk * _CHUNK
    anchor0 = chunk * _CHUNK

    pltpu.sync_copy(big_h.at[pl.ds(base, _CHUNK)], ac_v)
    pltpu.sync_copy(big_h.at[pl.ds(_SEG + base, _CHUNK)], aw_v)
    pltpu.sync_copy(big_h.at[pl.ds(2 * _SEG + base, _CHUNK)], l0_v)
    pltpu.sync_copy(big_h.at[pl.ds(3 * _SEG + base, _CHUNK)], l1_v)
    pltpu.sync_copy(big_h.at[pl.ds(4 * _SEG + base, _CHUNK)], igf_v)
    pltpu.sync_copy(big_h.at[pl.ds(5 * _SEG + batch * 48, 48)], t_v)

    lane = lax.iota(jnp.int32, 16)
    neg1 = jnp.full((16,), -1.0, jnp.float32)
    zeroi = jnp.zeros((16,), jnp.int32)

    @plsc.parallel_loop(0, _VPT, unroll=2)
    def _setup(i):
        sl = pl.ds(i * 16, 16)
        ac = ac_v[sl]
        aw = aw_v[sl]
        a_s = ac - aw / 2.0
        a_e = ac + aw / 2.0
        as_v[sl] = a_s
        ae_v[sl] = a_e
        alen_v[sl] = a_e - a_s
        bov_v[sl] = neg1
        bidx_v[sl] = zeroi

    def per_g(g, carry):
        pk_i, pk_x = carry
        gidx = jnp.broadcast_to(g, (16,))
        gsb = plsc.load_gather(t_v, [3 * gidx])
        geb = plsc.load_gather(t_v, [3 * gidx + 1])
        glenb = geb - gsb

        def inner(i, cr):
            gm, gi = cr
            sl = pl.ds(i * 16, 16)
            a_s = as_v[sl]
            a_e = ae_v[sl]
            alen = alen_v[sl]
            inter = jnp.maximum(jnp.minimum(geb, a_e) - jnp.maximum(gsb, a_s), 0.0)
            union = jnp.maximum(glenb + alen - inter, 1e-10)
            iou = inter / union
            bov = bov_v[sl]
            upd = iou > bov
            bov_v[sl] = jnp.where(upd, iou, bov)
            bidx_v[sl] = jnp.where(upd, gidx, bidx_v[sl])
            aidx = anchor0 + i * 16 + lane
            gu = iou > gm
            gm = jnp.where(gu, iou, gm)
            gi = jnp.where(gu, aidx, gi)
            return gm, gi

        gm, gi = plsc.parallel_loop(0, _VPT, 1, unroll=8,
                                    carry=(neg1, zeroi))(inner)
        m = jnp.max(gm)
        cand = jnp.where(gm == m, gi, jnp.int32(2 ** 30))
        mi = jnp.min(cand)
        lm = lane == g
        pk_i = jnp.where(lm, m, pk_i)
        pk_x = jnp.where(lm, mi, pk_x)
        return pk_i, pk_x

    pk_i, pk_x = lax.fori_loop(0, _NG, per_g,
                               (jnp.zeros((16,), jnp.float32), zeroi))

    stage_v[pl.ds(0, 16)] = pk_i
    stage_v[pl.ds(16, 16)] = lax.bitcast_convert_type(pk_x, jnp.float32)
    pltpu.sync_copy(stage_v, sh.at[pl.ds(s * 32, 32)])
    plsc.subcore_barrier()
    s0 = (s // 4) * 4
    pltpu.sync_copy(sh.at[pl.ds(s0 * 32, 32)], tmp_v)
    cur_i = tmp_v[pl.ds(0, 16)]
    cur_x = lax.bitcast_convert_type(tmp_v[pl.ds(16, 16)], jnp.int32)
    for cc in range(1, 4):
        pltpu.sync_copy(sh.at[pl.ds((s0 + cc) * 32, 32)], tmp_v)
        vi = tmp_v[pl.ds(0, 16)]
        vx = lax.bitcast_convert_type(tmp_v[pl.ds(16, 16)], jnp.int32)
        u = vi > cur_i
        cur_i = jnp.where(u, vi, cur_i)
        cur_x = jnp.where(u, vx, cur_x)
    plsc.subcore_barrier()

    bp_v[...] = cur_x
    two_f = jnp.full((16,), 2.0, jnp.float32)

    def override(g, _):
        gidx = jnp.broadcast_to(g, (16,))
        bpg = plsc.load_gather(bp_v, [gidx])
        loc = bpg - anchor0
        mask = jnp.logical_and(lane == g,
                               jnp.logical_and(loc >= 0, loc < _CHUNK))
        li = jnp.clip(loc, 0, _CHUNK - 1)
        plsc.store_scatter(bov_v, [li], two_f, mask=mask)
        plsc.store_scatter(bidx_v, [li], gidx, mask=mask)
        return 0

    lax.fori_loop(0, _NG, override, 0)

    gs_vec = plsc.load_gather(t_v, [3 * lane])
    ge_vec = plsc.load_gather(t_v, [3 * lane + 1])
    mc_v[...] = (gs_vec + ge_vec) / 2.0
    mw_v[...] = ge_vec - gs_vec

    def p2(i, carry):
        ls, cs = carry
        sl = pl.ds(i * 16, 16)
        bov = bov_v[sl]
        bidx = bidx_v[sl]
        igf = igf_v[sl]
        ac = ac_v[sl]
        aw = aw_v[sl]
        l0 = l0_v[sl]
        l1 = l1_v[sl]
        p = jnp.logical_and(bov >= 0.5, igf == 0.0)
        mc = plsc.load_gather(mc_v, [bidx])
        mw = plsc.load_gather(mw_v, [bidx])
        lc = (mc - ac) / (0.1 * aw)
        r = jnp.maximum(mw / aw, 1e-10)
        lw = _vlog(r) / 0.2
        d0 = l0 - lc
        d1 = l1 - lw
        a0 = jnp.abs(d0)
        a1 = jnp.abs(d1)
        s0_ = jnp.where(a0 < 1.0, 0.5 * a0 * a0, a0 - 0.5)
        s1_ = jnp.where(a1 < 1.0, 0.5 * a1 * a1, a1 - 0.5)
        ls = ls + jnp.where(p, s0_ + s1_, 0.0)
        cs = cs + jnp.where(p, 1.0, 0.0)
        return ls, cs

    zero16 = jnp.zeros((16,), jnp.float32)
    ls, cs = plsc.parallel_loop(0, _VPT, 1, unroll=2,
                                carry=(zero16, zero16))(p2)
    lsum = jnp.sum(ls)
    csum = jnp.sum(cs)
    stage_v[pl.ds(0, 16)] = jnp.where(lane == 0, lsum, jnp.where(lane == 1, csum, 0.0))
    pltpu.sync_copy(stage_v, sh.at[pl.ds(s * 32, 32)])
    plsc.subcore_barrier()

    @pl.when(s == 0)
    def _():
        acc = jnp.zeros((16,), jnp.float32)
        for k in range(16):
            pltpu.sync_copy(sh.at[pl.ds(k * 32, 32)], tmp_v)
            acc = acc + tmp_v[pl.ds(0, 16)]
        res_v[...] = acc
        pltpu.sync_copy(res_v, out_h.at[pl.ds(c * 16, 16)])


@functools.partial(
    pl.kernel,
    mesh=plsc.VectorSubcoreMesh(core_axis_name="c", subcore_axis_name="s"),
    out_type=jax.ShapeDtypeStruct((32,), jnp.float32),
    compiler_params=pltpu.CompilerParams(needs_layout_passes=False),
    scratch_types=[
        pltpu.VMEM((_CHUNK,), jnp.float32),
        pltpu.VMEM((_CHUNK,), jnp.float32),
        pltpu.VMEM((_CHUNK,), jnp.float32),
        pltpu.VMEM((_CHUNK,), jnp.float32),
        pltpu.VMEM((_CHUNK,), jnp.float32),
        pltpu.VMEM((3 * _NG,), jnp.float32),
        pltpu.VMEM((_CHUNK,), jnp.float32),
        pltpu.VMEM((_CHUNK,), jnp.float32),
        pltpu.VMEM((_CHUNK,), jnp.float32),
        pltpu.VMEM((_CHUNK,), jnp.float32),
        pltpu.VMEM((_CHUNK,), jnp.int32),
        pltpu.VMEM((16,), jnp.int32),
        pltpu.VMEM((32,), jnp.float32),
        pltpu.VMEM((32,), jnp.float32),
        pltpu.VMEM((_NG,), jnp.float32),
        pltpu.VMEM((_NG,), jnp.float32),
        pltpu.VMEM((16,), jnp.float32),
        pltpu.VMEM_SHARED((512,), jnp.float32),
    ],
)
def _fgd_sc(*refs):
    _sc_body(*refs)


def kernel(loc_pred, conf_pred, refined_anchors, ignore_flags_refined_anchor, targets):
    del conf_pred
    pad = _PAD_NA - _NA
    ac = jnp.pad(refined_anchors[..., 0], ((0, 0), (0, pad)), constant_values=-10.0)
    aw = jnp.pad(refined_anchors[..., 1], ((0, 0), (0, pad)), constant_values=1.0)
    l0 = jnp.pad(loc_pred[..., 0], ((0, 0), (0, pad)))
    l1 = jnp.pad(loc_pred[..., 1], ((0, 0), (0, pad)))
    igf = jnp.pad(ignore_flags_refined_anchor.astype(jnp.float32),
                  ((0, 0), (0, pad)), constant_values=1.0)
    big = jnp.concatenate([ac.reshape(-1), aw.reshape(-1), l0.reshape(-1),
                           l1.reshape(-1), igf.reshape(-1), targets.reshape(-1)])
    out = _fgd_sc(big)
    return (out[0] + out[16]) / (out[1] + out[17])
